# Initial kernel scaffold; baseline (speedup 1.0000x reference)
import jax
import jax.numpy as jnp
from jax.experimental import pallas as pl


def kernel(hidden_states, W, b):
    raise NotImplementedError("write your pallas kernel here")


# speedup vs baseline: 1.4963x; 1.4963x over previous
import jax
import jax.numpy as jnp
from jax import lax
from jax.experimental import pallas as pl

B, S, D = 4, 8192, 1024
BLK = 512
NBLK = S // BLK
K = max(1, int(0.5 * S))
ONE_BITS = 0x3F800000


def _body(hs_ref, w_ref, b_ref, weights_ref, mask_ref):
    i = pl.program_id(0)
    hs = hs_ref[...]
    w = w_ref[...]
    logits = jnp.dot(hs.reshape(B * BLK, D), w.reshape(D, 1),
                     preferred_element_type=jnp.float32)
    logits = logits.reshape(B, BLK) + b_ref[0, 0]
    weights_ref[:, pl.ds(i * BLK, BLK)] = jax.nn.sigmoid(logits)

    @pl.when(i == NBLK - 1)
    def _():
        u = lax.bitcast_convert_type(weights_ref[...], jnp.int32)
        lo0 = jnp.zeros((B, 1), jnp.int32)
        hi0 = jnp.full((B, 1), ONE_BITS, jnp.int32)

        def step(_, carry):
            lo, hi = carry
            mid = (lo + hi + 1) >> 1
            cnt = jnp.sum((u >= mid).astype(jnp.int32), axis=1, keepdims=True)
            ge = cnt >= K
            return jnp.where(ge, mid, lo), jnp.where(ge, hi, mid - 1)

        lo, _ = lax.fori_loop(0, 31, step, (lo0, hi0))
        mask_ref[...] = (u >= lo).astype(jnp.int32)


def kernel(hidden_states, W, b):
    weights, mask = pl.pallas_call(
        _body,
        grid=(NBLK,),
        in_specs=[
            pl.BlockSpec((B, BLK, D), lambda i: (0, i, 0)),
            pl.BlockSpec((1, D), lambda i: (0, 0)),
            pl.BlockSpec((1, 1), lambda i: (0, 0)),
        ],
        out_specs=[
            pl.BlockSpec((B, S), lambda i: (0, 0)),
            pl.BlockSpec((B, S), lambda i: (0, 0)),
        ],
        out_shape=[
            jax.ShapeDtypeStruct((B, S), jnp.float32),
            jax.ShapeDtypeStruct((B, S), jnp.int32),
        ],
    )(hidden_states, W, b.reshape(1, 1))
    return weights, mask.astype(bool)
